# baseline (speedup 1.0000x reference)
import jax
import jax.numpy as jnp
from jax import lax
from jax.experimental import pallas as pl
from jax.experimental.pallas import tpu as pltpu
from jax.experimental.pallas import tpu_sc as plsc

BATCH = 128
N_INPUT = 262144
N_MODULES = 256
H1 = 256
H2 = 128
N_OUT = 10
OUT_PAD = 128

_LN = 16
_C = 4096
_NWORK = 32
_RPW = BATCH // _NWORK
_SKEW = N_MODULES + 1
_ACC = _LN * _SKEW

_N_SC = 163840
_NCHUNK = _N_SC // _C
_N_TC = N_INPUT - _N_SC
_BLK = 2048
_NB = _N_TC // _BLK
_TC_OFF = _N_SC // _BLK


def _sc_body(x_hbm, fidx_hbm, w_hbm, out_hbm,
             fidx_buf, w_buf, x_buf, a0, a1, a2, a3, res_buf,
             fidx_sh, w_sh, xsem0, xsem1, lsem0, lsem1, sem_stage):
    cid = lax.axis_index("c")
    sid = lax.axis_index("s")
    wid = sid * 2 + cid
    row0 = wid * _RPW
    accs = (a0, a1, a2, a3)
    xsems = (xsem0, xsem1)
    lsems = (lsem0, lsem1)

    zero = jnp.zeros((_LN,), jnp.float32)

    @plsc.parallel_loop(0, _ACC // _LN, unroll=4)
    def _(i):
        for a in accs:
            a[pl.ds(i * _LN, _LN)] = zero

    def stage_copies():
        cs = []
        for kk in range(_NCHUNK):
            o = pl.ds(kk * _C, _C)
            cs.append(pltpu.make_async_copy(fidx_hbm.at[o], fidx_sh.at[o],
                                            sem_stage))
            cs.append(pltpu.make_async_copy(w_hbm.at[o], w_sh.at[o],
                                            sem_stage))
        return cs

    @pl.when(sid == 0)
    def _():
        start_list = stage_copies()
        for c in start_list:
            c.start()
        for c in stage_copies():
            c.wait()

    plsc.subcore_barrier()

    def local_copies(k, par):
        off = k * _C
        return [pltpu.make_async_copy(fidx_sh.at[pl.ds(off, _C)],
                                      fidx_buf.at[par], lsems[par]),
                pltpu.make_async_copy(w_sh.at[pl.ds(off, _C)],
                                      w_buf.at[par], lsems[par])]

    def x_copies(k, par):
        off = k * _C
        return [pltpu.make_async_copy(
            x_hbm.at[row0 + r, pl.ds(off, _C)], x_buf.at[par, r], xsems[par])
            for r in range(_RPW)]

    def start(cs):
        for c in cs:
            c.start()

    def wait(cs):
        for c in cs:
            c.wait()

    def compute(par):
        fv0 = fidx_buf[par, pl.ds(0, _LN)]
        carry0 = (fv0, zero, zero, zero, zero)

        @plsc.parallel_loop(0, _C // _LN, unroll=4, carry=carry0)
        def fin(i, carry):
            fv_cur = carry[0]
            avs = carry[1:]
            s = pl.ds(i * _LN, _LN)
            fv = fidx_buf[par, s]
            wv = w_buf[par, s]
            mask = fv != fv_cur
            outs = []
            for r in range(_RPW):
                plsc.addupdate_scatter(accs[r], [fv_cur], avs[r], mask=mask)
                xw = x_buf[par, r, s] * wv
                outs.append(jnp.where(mask, xw, avs[r] + xw))
            return (fv,) + tuple(outs)

        for r in range(_RPW):
            plsc.addupdate_scatter(accs[r], [fin[0]], fin[1 + r])

    start(local_copies(0, 0))
    start(x_copies(0, 0))

    def half_step(k, par):
        nxt = par ^ 1

        @pl.when(k + 1 < _NCHUNK)
        def _():
            start(local_copies(k + 1, nxt))
            start(x_copies(k + 1, nxt))

        wait(local_copies(0, par))
        wait(x_copies(0, par))
        compute(par)

    def outer(k2, carry):
        k = 2 * k2
        half_step(k, 0)
        half_step(k + 1, 1)
        return carry

    lax.fori_loop(0, _NCHUNK // 2, outer, 0)

    for r in range(_RPW):
        def red_body(j, carry, _acc=accs[r]):
            t = _acc[pl.ds(j * _LN, _LN)]
            for l in range(1, _LN):
                t = t + _acc[pl.ds(l * _SKEW + j * _LN, _LN)]
            res_buf[pl.ds(j * _LN, _LN)] = t
            return carry

        lax.fori_loop(0, N_MODULES // _LN, red_body, 0)
        pltpu.sync_copy(res_buf, out_hbm.at[row0 + r])


def _segment_sum_sc(x, idx, w):
    lane = (jnp.arange(_N_SC, dtype=jnp.int32) % _LN) * _SKEW
    fidx = lane + idx[:_N_SC].astype(jnp.int32)
    wf = w.reshape(-1)[:_N_SC]
    mesh = plsc.VectorSubcoreMesh(core_axis_name="c", subcore_axis_name="s")
    run = pl.kernel(
        _sc_body,
        mesh=mesh,
        compiler_params=pltpu.CompilerParams(needs_layout_passes=False),
        out_type=jax.ShapeDtypeStruct((BATCH, N_MODULES), jnp.float32),
        scratch_types=[
            pltpu.VMEM((2, _C), jnp.int32),
            pltpu.VMEM((2, _C), jnp.float32),
            pltpu.VMEM((2, _RPW, _C), jnp.float32),
            pltpu.VMEM((_ACC,), jnp.float32),
            pltpu.VMEM((_ACC,), jnp.float32),
            pltpu.VMEM((_ACC,), jnp.float32),
            pltpu.VMEM((_ACC,), jnp.float32),
            pltpu.VMEM((N_MODULES,), jnp.float32),
            pltpu.VMEM_SHARED((_N_SC,), jnp.int32),
            pltpu.VMEM_SHARED((_N_SC,), jnp.float32),
            pltpu.SemaphoreType.DMA,
            pltpu.SemaphoreType.DMA,
            pltpu.SemaphoreType.DMA,
            pltpu.SemaphoreType.DMA,
            pltpu.SemaphoreType.DMA,
        ],
    )
    return run(x, fidx, wf)


def _tc_seg_body(x_ref, idx_ref, w_ref, out_ref):
    j = pl.program_id(0)
    xw = x_ref[...] * w_ref[0, 0, :][None, :]
    iv = idx_ref[0, 0, :]
    oh = (iv[:, None] == jax.lax.broadcasted_iota(jnp.int32, (_BLK, N_MODULES), 1)
          ).astype(jnp.float32)
    part = jax.lax.dot_general(
        xw, oh, (((1,), (0,)), ((), ())),
        preferred_element_type=jnp.float32,
        precision=jax.lax.Precision.HIGHEST)

    @pl.when(j == 0)
    def _():
        out_ref[...] = jnp.zeros_like(out_ref)

    out_ref[...] += part


def _segment_sum_tc(x, idx, w):
    idx3 = idx[_N_SC:].astype(jnp.int32).reshape(_NB, 1, _BLK)
    w3 = w.reshape(-1)[_N_SC:].reshape(_NB, 1, _BLK)
    return pl.pallas_call(
        _tc_seg_body,
        grid=(_NB,),
        in_specs=[
            pl.BlockSpec((BATCH, _BLK), lambda j: (0, j + _TC_OFF)),
            pl.BlockSpec((1, 1, _BLK), lambda j: (j, 0, 0)),
            pl.BlockSpec((1, 1, _BLK), lambda j: (j, 0, 0)),
        ],
        out_specs=pl.BlockSpec((BATCH, N_MODULES), lambda j: (0, 0)),
        out_shape=jax.ShapeDtypeStruct((BATCH, N_MODULES), jnp.float32),
    )(x, idx3, w3)


def _bn(h, gamma, beta):
    mu = jnp.mean(h, axis=0, keepdims=True)
    var = jnp.mean((h - mu) ** 2, axis=0, keepdims=True)
    return gamma * (h - mu) * jax.lax.rsqrt(var + 1e-5) + beta


def _dot(a, b):
    return jax.lax.dot_general(a, b, (((1,), (0,)), ((), ())),
                               preferred_element_type=jnp.float32,
                               precision=jax.lax.Precision.HIGHEST)


def _head_body(wx_ref, wx2_ref, g0_ref, b0_ref, W1_ref, b1_ref, g1_ref, bb1_ref,
               W2_ref, b2_ref, g2_ref, bb2_ref, W3_ref, b3_ref,
               out_ref, z_ref):
    z = _bn(jnp.maximum(wx_ref[...] + wx2_ref[...], 0.0), g0_ref[...], b0_ref[...])
    z_ref[...] = z
    h = _bn(jnp.maximum(_dot(z, W1_ref[...]) + b1_ref[...], 0.0),
            g1_ref[...], bb1_ref[...])
    h = _bn(jnp.maximum(_dot(h, W2_ref[...]) + b2_ref[...], 0.0),
            g2_ref[...], bb2_ref[...])
    logits = _dot(h, W3_ref[...]) + b3_ref[...]
    col = jax.lax.broadcasted_iota(jnp.int32, (BATCH, OUT_PAD), 1)
    logits = jnp.where(col < N_OUT, logits, -1e30)
    m = jnp.max(logits, axis=-1, keepdims=True)
    e = jnp.exp(logits - m)
    out_ref[...] = e / jnp.sum(e, axis=-1, keepdims=True)


def _head(wx, wx2, g0, b0, W1, b1, g1, bb1, W2, b2, g2, bb2, W3, b3):
    W3p = jnp.zeros((H2, OUT_PAD), jnp.float32).at[:, :N_OUT].set(W3)
    b3p = jnp.zeros((1, OUT_PAD), jnp.float32).at[0, :N_OUT].set(b3)
    args = (wx, wx2, g0.reshape(1, -1), b0.reshape(1, -1), W1, b1.reshape(1, -1),
            g1.reshape(1, -1), bb1.reshape(1, -1), W2, b2.reshape(1, -1),
            g2.reshape(1, -1), bb2.reshape(1, -1), W3p, b3p)
    out, z = pl.pallas_call(
        _head_body,
        out_shape=(jax.ShapeDtypeStruct((BATCH, OUT_PAD), jnp.float32),
                   jax.ShapeDtypeStruct((BATCH, N_MODULES), jnp.float32)),
    )(*args)
    return out[:, :N_OUT], z


def kernel(x, idx, w, g0, b0, W1, b1, g1, bb1, W2, b2, g2, bb2, W3, b3):
    wx_sc = _segment_sum_sc(x, idx, w)
    wx_tc = _segment_sum_tc(x, idx, w)
    out, z = _head(wx_sc, wx_tc, g0, b0, W1, b1, g1, bb1, W2, b2, g2, bb2,
                   W3, b3)
    return (out, z)


# speedup vs baseline: 1.0384x; 1.0384x over previous
import jax
import jax.numpy as jnp
from jax import lax
from jax.experimental import pallas as pl
from jax.experimental.pallas import tpu as pltpu
from jax.experimental.pallas import tpu_sc as plsc

BATCH = 128
N_INPUT = 262144
N_MODULES = 256
H1 = 256
H2 = 128
N_OUT = 10
OUT_PAD = 128

_LN = 16
_C = 4096
_NWORK = 32
_RPW = BATCH // _NWORK
_SKEW = N_MODULES + 1
_ACC = _LN * _SKEW

_N_SC = 155648
_NCHUNK = _N_SC // _C
_N_TC = N_INPUT - _N_SC
_BLK = 2048
_NB = _N_TC // _BLK
_TC_OFF = _N_SC // _BLK


def _sc_body(x_hbm, fidx_hbm, w_hbm, out_hbm,
             fidx_buf, w_buf, x_buf, a0, a1, a2, a3, res_buf,
             fidx_sh, w_sh, xsem0, xsem1, lsem0, lsem1, sem_stage):
    cid = lax.axis_index("c")
    sid = lax.axis_index("s")
    wid = sid * 2 + cid
    row0 = wid * _RPW
    accs = (a0, a1, a2, a3)
    xsems = (xsem0, xsem1)
    lsems = (lsem0, lsem1)

    zero = jnp.zeros((_LN,), jnp.float32)

    @plsc.parallel_loop(0, _ACC // _LN, unroll=4)
    def _(i):
        for a in accs:
            a[pl.ds(i * _LN, _LN)] = zero

    def stage_copies():
        cs = []
        for kk in range(_NCHUNK):
            o = pl.ds(kk * _C, _C)
            cs.append(pltpu.make_async_copy(fidx_hbm.at[o], fidx_sh.at[o],
                                            sem_stage))
            cs.append(pltpu.make_async_copy(w_hbm.at[o], w_sh.at[o],
                                            sem_stage))
        return cs

    @pl.when(sid == 0)
    def _():
        start_list = stage_copies()
        for c in start_list:
            c.start()
        for c in stage_copies():
            c.wait()

    plsc.subcore_barrier()

    def local_copies(k, par):
        off = k * _C
        return [pltpu.make_async_copy(fidx_sh.at[pl.ds(off, _C)],
                                      fidx_buf.at[par], lsems[par]),
                pltpu.make_async_copy(w_sh.at[pl.ds(off, _C)],
                                      w_buf.at[par], lsems[par])]

    def x_copies(k, par):
        off = k * _C
        return [pltpu.make_async_copy(
            x_hbm.at[row0 + r, pl.ds(off, _C)], x_buf.at[par, r], xsems[par])
            for r in range(_RPW)]

    def start(cs):
        for c in cs:
            c.start()

    def wait(cs):
        for c in cs:
            c.wait()

    def compute(par):
        fv0 = fidx_buf[par, pl.ds(0, _LN)]
        carry0 = (fv0, zero, zero, zero, zero)

        @plsc.parallel_loop(0, _C // _LN, unroll=4, carry=carry0)
        def fin(i, carry):
            fv_cur = carry[0]
            avs = carry[1:]
            s = pl.ds(i * _LN, _LN)
            fv = fidx_buf[par, s]
            wv = w_buf[par, s]
            mask = fv != fv_cur
            outs = []
            for r in range(_RPW):
                plsc.addupdate_scatter(accs[r], [fv_cur], avs[r], mask=mask)
                xw = x_buf[par, r, s] * wv
                outs.append(jnp.where(mask, xw, avs[r] + xw))
            return (fv,) + tuple(outs)

        for r in range(_RPW):
            plsc.addupdate_scatter(accs[r], [fin[0]], fin[1 + r])

    start(local_copies(0, 0))
    start(x_copies(0, 0))

    def half_step(k, par):
        nxt = par ^ 1

        @pl.when(k + 1 < _NCHUNK)
        def _():
            start(local_copies(k + 1, nxt))
            start(x_copies(k + 1, nxt))

        wait(local_copies(0, par))
        wait(x_copies(0, par))
        compute(par)

    def outer(k2, carry):
        k = 2 * k2
        half_step(k, 0)
        half_step(k + 1, 1)
        return carry

    lax.fori_loop(0, _NCHUNK // 2, outer, 0)

    for r in range(_RPW):
        def red_body(j, carry, _acc=accs[r]):
            t = _acc[pl.ds(j * _LN, _LN)]
            for l in range(1, _LN):
                t = t + _acc[pl.ds(l * _SKEW + j * _LN, _LN)]
            res_buf[pl.ds(j * _LN, _LN)] = t
            return carry

        lax.fori_loop(0, N_MODULES // _LN, red_body, 0)
        pltpu.sync_copy(res_buf, out_hbm.at[row0 + r])


def _segment_sum_sc(x, idx, w):
    lane = (jnp.arange(_N_SC, dtype=jnp.int32) % _LN) * _SKEW
    fidx = lane + idx[:_N_SC].astype(jnp.int32)
    wf = w.reshape(-1)[:_N_SC]
    mesh = plsc.VectorSubcoreMesh(core_axis_name="c", subcore_axis_name="s")
    run = pl.kernel(
        _sc_body,
        mesh=mesh,
        compiler_params=pltpu.CompilerParams(needs_layout_passes=False),
        out_type=jax.ShapeDtypeStruct((BATCH, N_MODULES), jnp.float32),
        scratch_types=[
            pltpu.VMEM((2, _C), jnp.int32),
            pltpu.VMEM((2, _C), jnp.float32),
            pltpu.VMEM((2, _RPW, _C), jnp.float32),
            pltpu.VMEM((_ACC,), jnp.float32),
            pltpu.VMEM((_ACC,), jnp.float32),
            pltpu.VMEM((_ACC,), jnp.float32),
            pltpu.VMEM((_ACC,), jnp.float32),
            pltpu.VMEM((N_MODULES,), jnp.float32),
            pltpu.VMEM_SHARED((_N_SC,), jnp.int32),
            pltpu.VMEM_SHARED((_N_SC,), jnp.float32),
            pltpu.SemaphoreType.DMA,
            pltpu.SemaphoreType.DMA,
            pltpu.SemaphoreType.DMA,
            pltpu.SemaphoreType.DMA,
            pltpu.SemaphoreType.DMA,
        ],
    )
    return run(x, fidx, wf)


def _tc_seg_body(x_ref, idx_ref, w_ref, out_ref):
    j = pl.program_id(0)
    xw = x_ref[...] * w_ref[0, 0, :][None, :]
    iv = idx_ref[0, 0, :]
    oh = (iv[:, None] == jax.lax.broadcasted_iota(jnp.int32, (_BLK, N_MODULES), 1)
          ).astype(jnp.float32)
    part = jax.lax.dot_general(
        xw, oh, (((1,), (0,)), ((), ())),
        preferred_element_type=jnp.float32,
        precision=jax.lax.Precision.HIGHEST)

    @pl.when(j == 0)
    def _():
        out_ref[...] = jnp.zeros_like(out_ref)

    out_ref[...] += part


def _segment_sum_tc(x, idx, w):
    idx3 = idx[_N_SC:].astype(jnp.int32).reshape(_NB, 1, _BLK)
    w3 = w.reshape(-1)[_N_SC:].reshape(_NB, 1, _BLK)
    return pl.pallas_call(
        _tc_seg_body,
        grid=(_NB,),
        in_specs=[
            pl.BlockSpec((BATCH, _BLK), lambda j: (0, j + _TC_OFF)),
            pl.BlockSpec((1, 1, _BLK), lambda j: (j, 0, 0)),
            pl.BlockSpec((1, 1, _BLK), lambda j: (j, 0, 0)),
        ],
        out_specs=pl.BlockSpec((BATCH, N_MODULES), lambda j: (0, 0)),
        out_shape=jax.ShapeDtypeStruct((BATCH, N_MODULES), jnp.float32),
    )(x, idx3, w3)


def _bn(h, gamma, beta):
    mu = jnp.mean(h, axis=0, keepdims=True)
    var = jnp.mean((h - mu) ** 2, axis=0, keepdims=True)
    return gamma * (h - mu) * jax.lax.rsqrt(var + 1e-5) + beta


def _dot(a, b):
    return jax.lax.dot_general(a, b, (((1,), (0,)), ((), ())),
                               preferred_element_type=jnp.float32,
                               precision=jax.lax.Precision.HIGHEST)


def _head_body(wx_ref, wx2_ref, g0_ref, b0_ref, W1_ref, b1_ref, g1_ref, bb1_ref,
               W2_ref, b2_ref, g2_ref, bb2_ref, W3_ref, b3_ref,
               out_ref, z_ref):
    z = _bn(jnp.maximum(wx_ref[...] + wx2_ref[...], 0.0), g0_ref[...], b0_ref[...])
    z_ref[...] = z
    h = _bn(jnp.maximum(_dot(z, W1_ref[...]) + b1_ref[...], 0.0),
            g1_ref[...], bb1_ref[...])
    h = _bn(jnp.maximum(_dot(h, W2_ref[...]) + b2_ref[...], 0.0),
            g2_ref[...], bb2_ref[...])
    logits = _dot(h, W3_ref[...]) + b3_ref[...]
    col = jax.lax.broadcasted_iota(jnp.int32, (BATCH, OUT_PAD), 1)
    logits = jnp.where(col < N_OUT, logits, -1e30)
    m = jnp.max(logits, axis=-1, keepdims=True)
    e = jnp.exp(logits - m)
    out_ref[...] = e / jnp.sum(e, axis=-1, keepdims=True)


def _head(wx, wx2, g0, b0, W1, b1, g1, bb1, W2, b2, g2, bb2, W3, b3):
    W3p = jnp.zeros((H2, OUT_PAD), jnp.float32).at[:, :N_OUT].set(W3)
    b3p = jnp.zeros((1, OUT_PAD), jnp.float32).at[0, :N_OUT].set(b3)
    args = (wx, wx2, g0.reshape(1, -1), b0.reshape(1, -1), W1, b1.reshape(1, -1),
            g1.reshape(1, -1), bb1.reshape(1, -1), W2, b2.reshape(1, -1),
            g2.reshape(1, -1), bb2.reshape(1, -1), W3p, b3p)
    out, z = pl.pallas_call(
        _head_body,
        out_shape=(jax.ShapeDtypeStruct((BATCH, OUT_PAD), jnp.float32),
                   jax.ShapeDtypeStruct((BATCH, N_MODULES), jnp.float32)),
    )(*args)
    return out[:, :N_OUT], z


def kernel(x, idx, w, g0, b0, W1, b1, g1, bb1, W2, b2, g2, bb2, W3, b3):
    wx_sc = _segment_sum_sc(x, idx, w)
    wx_tc = _segment_sum_tc(x, idx, w)
    out, z = _head(wx_sc, wx_tc, g0, b0, W1, b1, g1, bb1, W2, b2, g2, bb2,
                   W3, b3)
    return (out, z)
